# baseline (speedup 1.0000x reference)
import functools

import jax
import jax.numpy as jnp
from jax import lax
from jax.experimental import pallas as pl
from jax.experimental.pallas import tpu as pltpu
from jax.experimental.pallas import tpu_sc as plsc

N = 8192
C = 2048
L = 16
NUM_CORES = 2
NUM_SUBCORES = 16
NW = NUM_CORES * NUM_SUBCORES
ROWS_PER_W = N // NW
GROUPS = ROWS_PER_W // L
U = 8
GL = L * C
NB = 256
HL = L * NB
SCAN_U = 4


def _body(x_hbm, out_hbm, buf0, buf1, buf_t, hcnt, hsum, in_sem):
    cid = lax.axis_index("c")
    sid = lax.axis_index("s")
    wid = sid * NUM_CORES + cid
    rowbase = lax.iota(jnp.int32, L) * C
    histbase = lax.iota(jnp.int32, L) * NB
    ones = jnp.ones((L,), jnp.float32)
    zeros = jnp.zeros((L,), jnp.float32)

    def src(g):
        base = (wid * GROUPS + lax.rem(g, GROUPS)) * GL
        return x_hbm.at[pl.ds(base, GL)]

    @plsc.parallel_loop(0, NB, step=U)
    def zero_loop(j):
        for u in range(U):
            hcnt[pl.ds((j + u) * L, L)] = zeros
            hsum[pl.ds((j + u) * L, L)] = zeros

    pltpu.async_copy(src(0), buf0, in_sem)

    def do_group(g, buf):
        base = (wid * GROUPS + g) * GL

        @plsc.parallel_loop(0, C, step=U,
                            carry=((jnp.full((L,), -1e30, jnp.float32),) * U,
                                   rowbase))
        def trans_loop(j, st):
            ms, idx0 = st
            out = []
            for u in range(U):
                v = plsc.load_gather(buf, [idx0 + u])
                buf_t[pl.ds((j + u) * L, L)] = v
                out.append(jnp.maximum(ms[u], v))
            return tuple(out), idx0 + U

        m = functools.reduce(jnp.maximum, trans_loop[0])

        @plsc.parallel_loop(0, GL, step=U * L)
        def hist_loop(o):
            for u in range(U):
                v = buf_t[pl.ds(o + u * L, L)]
                e = (m - v) * 256.0
                msk = e < 256.0
                idx = histbase + e.astype(jnp.int32)
                plsc.addupdate_scatter(hcnt, [idx], ones, mask=msk)
                plsc.addupdate_scatter(hsum, [idx], v, mask=msk)

        def scan_step(i, st):
            idx, tj, Kc, Sc, Kat, Sat, found = st
            for _ in range(SCAN_U):
                Kc = Kc + plsc.load_gather(hcnt, [idx])
                Sc = Sc + plsc.load_gather(hsum, [idx])
                plsc.store_scatter(hcnt, [idx], zeros)
                plsc.store_scatter(hsum, [idx], zeros)
                cross = jnp.logical_and(Sc - tj * Kc >= 1.0,
                                        jnp.logical_not(found))
                Kat = jnp.where(cross, Kc, Kat)
                Sat = jnp.where(cross, Sc, Sat)
                found = jnp.logical_or(found, cross)
                idx = idx + 1
                tj = tj - (1.0 / 256.0)
            return idx, tj, Kc, Sc, Kat, Sat, found

        _, _, _, _, Kat, Sat, _ = lax.fori_loop(
            0, NB // SCAN_U, scan_step,
            (histbase, m - (1.0 / 256.0), zeros, zeros, ones, m,
             jnp.zeros((L,), jnp.bool_)))
        tau = (Sat - 1.0) / Kat

        for _ in range(2):
            t = tau

            @plsc.parallel_loop(0, GL, step=U * L,
                                carry=(zeros,) * (2 * U))
            def ks_loop(o, ks):
                ks = list(ks)
                for u in range(U):
                    v = buf_t[pl.ds(o + u * L, L)]
                    act = v > t
                    ks[2 * u] = ks[2 * u] + jnp.where(act, 1.0, 0.0)
                    ks[2 * u + 1] = ks[2 * u + 1] + jnp.where(act, v, 0.0)
                return tuple(ks)

            k = functools.reduce(jnp.add, ks_loop[0::2])
            s = functools.reduce(jnp.add, ks_loop[1::2])
            tau = (s - 1.0) / k

        @plsc.parallel_loop(0, C, step=U, carry=rowbase)
        def out_loop(j, idx0):
            for u in range(U):
                v = jnp.maximum(buf_t[pl.ds((j + u) * L, L)] - tau, 0.0)
                plsc.store_scatter(buf, [idx0 + u], v)
            return idx0 + U

        pltpu.sync_copy(buf, out_hbm.at[pl.ds(base, GL)])

    def do_pair(h, carry):
        g0 = 2 * h
        pltpu.make_async_copy(src(g0), buf0, in_sem).wait()
        pltpu.async_copy(src(g0 + 1), buf1, in_sem)
        do_group(g0, buf0)
        pltpu.make_async_copy(src(g0 + 1), buf1, in_sem).wait()
        pltpu.async_copy(src(g0 + 2), buf0, in_sem)
        do_group(g0 + 1, buf1)
        return carry

    lax.fori_loop(0, GROUPS // 2, do_pair, 0)
    pltpu.make_async_copy(src(0), buf0, in_sem).wait()


_sparsemax_sc = functools.partial(
    pl.kernel,
    out_type=jax.ShapeDtypeStruct((N * C,), jnp.float32),
    mesh=plsc.VectorSubcoreMesh(
        core_axis_name="c", subcore_axis_name="s",
        num_cores=NUM_CORES, num_subcores=NUM_SUBCORES),
    scratch_types=[
        pltpu.VMEM((GL,), jnp.float32),
        pltpu.VMEM((GL,), jnp.float32),
        pltpu.VMEM((GL,), jnp.float32),
        pltpu.VMEM((HL,), jnp.float32),
        pltpu.VMEM((HL,), jnp.float32),
        pltpu.SemaphoreType.DMA,
    ],
    compiler_params=pltpu.CompilerParams(
        use_tc_tiling_on_sc=False, needs_layout_passes=False),
)(_body)


def kernel(input):
    return _sparsemax_sc(input.reshape(N * C)).reshape(N, C)


# speedup vs baseline: 1.0579x; 1.0579x over previous
import functools

import jax
import jax.numpy as jnp
from jax import lax
from jax.experimental import pallas as pl
from jax.experimental.pallas import tpu as pltpu
from jax.experimental.pallas import tpu_sc as plsc

N = 8192
C = 2048
L = 16
NUM_CORES = 2
NUM_SUBCORES = 16
NW = NUM_CORES * NUM_SUBCORES
ROWS_PER_W = N // NW
GROUPS = ROWS_PER_W // L
U = 8
GL = L * C
NB = 256
HL = L * NB
SCAN_U = 4


def _body(x_hbm, out_hbm, buf, buf_t, hcnt, hsum):
    cid = lax.axis_index("c")
    sid = lax.axis_index("s")
    wid = sid * NUM_CORES + cid
    rowbase = lax.iota(jnp.int32, L) * C
    histbase = lax.iota(jnp.int32, L) * NB
    ones = jnp.ones((L,), jnp.float32)
    zeros = jnp.zeros((L,), jnp.float32)

    def do_group(g, carry):
        base = (wid * GROUPS + g) * GL
        pltpu.sync_copy(x_hbm.at[pl.ds(base, GL)], buf)

        @plsc.parallel_loop(0, C, step=U,
                            carry=((jnp.full((L,), -1e30, jnp.float32),) * U,
                                   rowbase))
        def trans_loop(j, st):
            ms, idx0 = st
            out = []
            for u in range(U):
                v = plsc.load_gather(buf, [idx0 + u])
                buf_t[pl.ds((j + u) * L, L)] = v
                out.append(jnp.maximum(ms[u], v))
            return tuple(out), idx0 + U

        m = functools.reduce(jnp.maximum, trans_loop[0])

        @plsc.parallel_loop(0, NB, step=U)
        def zero_loop(j):
            for u in range(U):
                hcnt[pl.ds((j + u) * L, L)] = zeros
                hsum[pl.ds((j + u) * L, L)] = zeros

        @plsc.parallel_loop(0, GL, step=U * L)
        def hist_loop(o):
            for u in range(U):
                v = buf_t[pl.ds(o + u * L, L)]
                e = (m - v) * 256.0
                msk = e < 256.0
                idx = histbase + e.astype(jnp.int32)
                plsc.addupdate_scatter(hcnt, [idx], ones, mask=msk)
                plsc.addupdate_scatter(hsum, [idx], v, mask=msk)

        def scan_step(i, st):
            idx, tj, Kc, Sc, Kat, Sat, found = st
            for _ in range(SCAN_U):
                Kc = Kc + plsc.load_gather(hcnt, [idx])
                Sc = Sc + plsc.load_gather(hsum, [idx])
                cross = jnp.logical_and(Sc - tj * Kc >= 1.0,
                                        jnp.logical_not(found))
                Kat = jnp.where(cross, Kc, Kat)
                Sat = jnp.where(cross, Sc, Sat)
                found = jnp.logical_or(found, cross)
                idx = idx + 1
                tj = tj - (1.0 / 256.0)
            return idx, tj, Kc, Sc, Kat, Sat, found

        _, _, _, _, Kat, Sat, _ = lax.fori_loop(
            0, NB // SCAN_U, scan_step,
            (histbase, m - (1.0 / 256.0), zeros, zeros, ones, m,
             jnp.zeros((L,), jnp.bool_)))
        tau = (Sat - 1.0) / Kat

        for _ in range(2):
            t = tau

            @plsc.parallel_loop(0, GL, step=U * L,
                                carry=(zeros,) * (2 * U))
            def ks_loop(o, ks):
                ks = list(ks)
                for u in range(U):
                    v = buf_t[pl.ds(o + u * L, L)]
                    act = v > t
                    ks[2 * u] = ks[2 * u] + jnp.where(act, 1.0, 0.0)
                    ks[2 * u + 1] = ks[2 * u + 1] + jnp.where(act, v, 0.0)
                return tuple(ks)

            k = functools.reduce(jnp.add, ks_loop[0::2])
            s = functools.reduce(jnp.add, ks_loop[1::2])
            tau = (s - 1.0) / k

        @plsc.parallel_loop(0, C, step=U, carry=rowbase)
        def out_loop(j, idx0):
            for u in range(U):
                v = jnp.maximum(buf_t[pl.ds((j + u) * L, L)] - tau, 0.0)
                plsc.store_scatter(buf, [idx0 + u], v)
            return idx0 + U

        pltpu.sync_copy(buf, out_hbm.at[pl.ds(base, GL)])
        return carry

    lax.fori_loop(0, GROUPS, do_group, 0)


_sparsemax_sc = functools.partial(
    pl.kernel,
    out_type=jax.ShapeDtypeStruct((N * C,), jnp.float32),
    mesh=plsc.VectorSubcoreMesh(
        core_axis_name="c", subcore_axis_name="s",
        num_cores=NUM_CORES, num_subcores=NUM_SUBCORES),
    scratch_types=[
        pltpu.VMEM((GL,), jnp.float32),
        pltpu.VMEM((GL,), jnp.float32),
        pltpu.VMEM((HL,), jnp.float32),
        pltpu.VMEM((HL,), jnp.float32),
    ],
    compiler_params=pltpu.CompilerParams(
        use_tc_tiling_on_sc=False, needs_layout_passes=False),
)(_body)


def kernel(input):
    return _sparsemax_sc(input.reshape(N * C)).reshape(N, C)


# speedup vs baseline: 1.0985x; 1.0384x over previous
import functools

import jax
import jax.numpy as jnp
from jax import lax
from jax.experimental import pallas as pl
from jax.experimental.pallas import tpu as pltpu
from jax.experimental.pallas import tpu_sc as plsc

N = 8192
C = 2048
L = 16
NUM_CORES = 2
NUM_SUBCORES = 16
NW = NUM_CORES * NUM_SUBCORES
ROWS_PER_W = N // NW
GROUPS = ROWS_PER_W // L
U = 8
GL = L * C
NB = 256
HL = L * NB
SCAN_U = 4


def _body(x_hbm, out_hbm, buf0, buf1, buf_t, hcnt, hsum, in_sem):
    cid = lax.axis_index("c")
    sid = lax.axis_index("s")
    wid = sid * NUM_CORES + cid
    rowbase = lax.iota(jnp.int32, L) * C
    histbase = lax.iota(jnp.int32, L) * NB
    ones = jnp.ones((L,), jnp.float32)
    zeros = jnp.zeros((L,), jnp.float32)

    def src(g):
        base = (wid * GROUPS + lax.rem(g, GROUPS)) * GL
        return x_hbm.at[pl.ds(base, GL)]

    pltpu.async_copy(src(0), buf0, in_sem)

    def do_group(g, buf):
        base = (wid * GROUPS + g) * GL

        @plsc.parallel_loop(0, C, step=U,
                            carry=((jnp.full((L,), -1e30, jnp.float32),) * U,
                                   rowbase))
        def trans_loop(j, st):
            ms, idx0 = st
            out = []
            for u in range(U):
                v = plsc.load_gather(buf, [idx0 + u])
                buf_t[pl.ds((j + u) * L, L)] = v
                out.append(jnp.maximum(ms[u], v))
            return tuple(out), idx0 + U

        m = functools.reduce(jnp.maximum, trans_loop[0])

        @plsc.parallel_loop(0, NB, step=U)
        def zero_loop(j):
            for u in range(U):
                hcnt[pl.ds((j + u) * L, L)] = zeros
                hsum[pl.ds((j + u) * L, L)] = zeros

        @plsc.parallel_loop(0, GL, step=U * L)
        def hist_loop(o):
            for u in range(U):
                v = buf_t[pl.ds(o + u * L, L)]
                e = (m - v) * 256.0
                msk = e < 256.0
                idx = histbase + e.astype(jnp.int32)
                plsc.addupdate_scatter(hcnt, [idx], ones, mask=msk)
                plsc.addupdate_scatter(hsum, [idx], v, mask=msk)

        def scan_step(i, st):
            idx, tj, Kc, Sc, Kat, Sat, found = st
            for _ in range(SCAN_U):
                Kc = Kc + plsc.load_gather(hcnt, [idx])
                Sc = Sc + plsc.load_gather(hsum, [idx])
                cross = jnp.logical_and(Sc - tj * Kc >= 1.0,
                                        jnp.logical_not(found))
                Kat = jnp.where(cross, Kc, Kat)
                Sat = jnp.where(cross, Sc, Sat)
                found = jnp.logical_or(found, cross)
                idx = idx + 1
                tj = tj - (1.0 / 256.0)
            return idx, tj, Kc, Sc, Kat, Sat, found

        _, _, _, _, Kat, Sat, _ = lax.fori_loop(
            0, NB // SCAN_U, scan_step,
            (histbase, m - (1.0 / 256.0), zeros, zeros, ones, m,
             jnp.zeros((L,), jnp.bool_)))
        tau = (Sat - 1.0) / Kat

        for _ in range(2):
            t = tau

            @plsc.parallel_loop(0, GL, step=U * L,
                                carry=(zeros,) * (2 * U))
            def ks_loop(o, ks):
                ks = list(ks)
                for u in range(U):
                    v = buf_t[pl.ds(o + u * L, L)]
                    act = v > t
                    ks[2 * u] = ks[2 * u] + jnp.where(act, 1.0, 0.0)
                    ks[2 * u + 1] = ks[2 * u + 1] + jnp.where(act, v, 0.0)
                return tuple(ks)

            k = functools.reduce(jnp.add, ks_loop[0::2])
            s = functools.reduce(jnp.add, ks_loop[1::2])
            tau = (s - 1.0) / k

        @plsc.parallel_loop(0, C, step=U, carry=rowbase)
        def out_loop(j, idx0):
            for u in range(U):
                v = jnp.maximum(buf_t[pl.ds((j + u) * L, L)] - tau, 0.0)
                plsc.store_scatter(buf, [idx0 + u], v)
            return idx0 + U

        pltpu.sync_copy(buf, out_hbm.at[pl.ds(base, GL)])

    def do_pair(h, carry):
        g0 = 2 * h
        pltpu.make_async_copy(src(g0), buf0, in_sem).wait()
        pltpu.async_copy(src(g0 + 1), buf1, in_sem)
        do_group(g0, buf0)
        pltpu.make_async_copy(src(g0 + 1), buf1, in_sem).wait()
        pltpu.async_copy(src(g0 + 2), buf0, in_sem)
        do_group(g0 + 1, buf1)
        return carry

    lax.fori_loop(0, GROUPS // 2, do_pair, 0)
    pltpu.make_async_copy(src(0), buf0, in_sem).wait()


_sparsemax_sc = functools.partial(
    pl.kernel,
    out_type=jax.ShapeDtypeStruct((N * C,), jnp.float32),
    mesh=plsc.VectorSubcoreMesh(
        core_axis_name="c", subcore_axis_name="s",
        num_cores=NUM_CORES, num_subcores=NUM_SUBCORES),
    scratch_types=[
        pltpu.VMEM((GL,), jnp.float32),
        pltpu.VMEM((GL,), jnp.float32),
        pltpu.VMEM((GL,), jnp.float32),
        pltpu.VMEM((HL,), jnp.float32),
        pltpu.VMEM((HL,), jnp.float32),
        pltpu.SemaphoreType.DMA,
    ],
    compiler_params=pltpu.CompilerParams(
        use_tc_tiling_on_sc=False, needs_layout_passes=False),
)(_body)


def kernel(input):
    return _sparsemax_sc(input.reshape(N * C)).reshape(N, C)


# speedup vs baseline: 2.1876x; 1.9914x over previous
import functools

import jax
import jax.numpy as jnp
from jax import lax
from jax.experimental import pallas as pl
from jax.experimental.pallas import tpu as pltpu
from jax.experimental.pallas import tpu_sc as plsc

N = 8192
C = 2048
L = 16
NUM_CORES = 2
NUM_SUBCORES = 16
NW = NUM_CORES * NUM_SUBCORES
ROWS_PER_W = N // NW
GROUPS = ROWS_PER_W // L
U = 8
GL = L * C
PL = L + 1
TL = C * PL
NB = 256
PB = NB + 1
HL = L * PB
SCAN_U = 4


def _body(x_hbm, out_hbm, buf0, buf1, buf_t, hcnt, hsum, in_sem):
    cid = lax.axis_index("c")
    sid = lax.axis_index("s")
    wid = sid * NUM_CORES + cid
    colbase = lax.iota(jnp.int32, L) * PL
    histbase = lax.iota(jnp.int32, L) * PB
    ones = jnp.ones((L,), jnp.float32)
    zeros = jnp.zeros((L,), jnp.float32)

    def src(g):
        base = (wid * GROUPS + lax.rem(g, GROUPS)) * GL
        return x_hbm.at[pl.ds(base, GL)]

    pltpu.async_copy(src(0), buf0, in_sem)

    def do_group(g, buf):
        base = (wid * GROUPS + g) * GL

        @plsc.parallel_loop(0, C, step=L, carry=colbase)
        def trans_loop(j, idx0):
            for l in range(L):
                v = buf[pl.ds(l * C + j, L)]
                plsc.store_scatter(buf_t, [idx0 + l], v)
            return idx0 + L * PL

        @plsc.parallel_loop(0, TL, step=U * PL,
                            carry=(jnp.full((L,), -1e30, jnp.float32),) * U)
        def max_loop(o, ms):
            return tuple(
                jnp.maximum(ms[u], buf_t[pl.ds(o + u * PL, L)])
                for u in range(U))

        m = functools.reduce(jnp.maximum, max_loop)

        @plsc.parallel_loop(0, NB, step=U)
        def zero_loop(j):
            for u in range(U):
                hcnt[pl.ds((j + u) * L, L)] = zeros
                hsum[pl.ds((j + u) * L, L)] = zeros

        @plsc.parallel_loop(0, TL, step=U * PL)
        def hist_loop(o):
            for u in range(U):
                v = buf_t[pl.ds(o + u * PL, L)]
                e = (m - v) * 256.0
                msk = e < 256.0
                idx = histbase + e.astype(jnp.int32)
                plsc.addupdate_scatter(hcnt, [idx], ones, mask=msk)
                plsc.addupdate_scatter(hsum, [idx], v, mask=msk)

        def scan_step(i, st):
            idx, tj, Kc, Sc, Kat, Sat, found = st
            for _ in range(SCAN_U):
                Kc = Kc + plsc.load_gather(hcnt, [idx])
                Sc = Sc + plsc.load_gather(hsum, [idx])
                cross = jnp.logical_and(Sc - tj * Kc >= 1.0,
                                        jnp.logical_not(found))
                Kat = jnp.where(cross, Kc, Kat)
                Sat = jnp.where(cross, Sc, Sat)
                found = jnp.logical_or(found, cross)
                idx = idx + 1
                tj = tj - (1.0 / 256.0)
            return idx, tj, Kc, Sc, Kat, Sat, found

        _, _, _, _, Kat, Sat, _ = lax.fori_loop(
            0, NB // SCAN_U, scan_step,
            (histbase, m - (1.0 / 256.0), zeros, zeros, ones, m,
             jnp.zeros((L,), jnp.bool_)))
        tau = (Sat - 1.0) / Kat

        for _ in range(2):
            t = tau

            @plsc.parallel_loop(0, TL, step=U * PL,
                                carry=(zeros,) * (2 * U))
            def ks_loop(o, ks):
                ks = list(ks)
                for u in range(U):
                    v = buf_t[pl.ds(o + u * PL, L)]
                    act = v > t
                    ks[2 * u] = ks[2 * u] + jnp.where(act, 1.0, 0.0)
                    ks[2 * u + 1] = ks[2 * u + 1] + jnp.where(act, v, 0.0)
                return tuple(ks)

            k = functools.reduce(jnp.add, ks_loop[0::2])
            s = functools.reduce(jnp.add, ks_loop[1::2])
            tau = (s - 1.0) / k

        @plsc.parallel_loop(0, TL, step=U * PL)
        def relu_loop(o):
            for u in range(U):
                sl = pl.ds(o + u * PL, L)
                buf_t[sl] = jnp.maximum(buf_t[sl] - tau, 0.0)

        @plsc.parallel_loop(0, C, step=L, carry=colbase)
        def out_loop(j, idx0):
            for l in range(L):
                buf[pl.ds(l * C + j, L)] = plsc.load_gather(buf_t, [idx0 + l])
            return idx0 + L * PL

        pltpu.sync_copy(buf, out_hbm.at[pl.ds(base, GL)])

    def do_pair(h, carry):
        g0 = 2 * h
        pltpu.make_async_copy(src(g0), buf0, in_sem).wait()
        pltpu.async_copy(src(g0 + 1), buf1, in_sem)
        do_group(g0, buf0)
        pltpu.make_async_copy(src(g0 + 1), buf1, in_sem).wait()
        pltpu.async_copy(src(g0 + 2), buf0, in_sem)
        do_group(g0 + 1, buf1)
        return carry

    lax.fori_loop(0, GROUPS // 2, do_pair, 0)
    pltpu.make_async_copy(src(0), buf0, in_sem).wait()


_sparsemax_sc = functools.partial(
    pl.kernel,
    out_type=jax.ShapeDtypeStruct((N * C,), jnp.float32),
    mesh=plsc.VectorSubcoreMesh(
        core_axis_name="c", subcore_axis_name="s",
        num_cores=NUM_CORES, num_subcores=NUM_SUBCORES),
    scratch_types=[
        pltpu.VMEM((GL,), jnp.float32),
        pltpu.VMEM((GL,), jnp.float32),
        pltpu.VMEM((TL,), jnp.float32),
        pltpu.VMEM((HL,), jnp.float32),
        pltpu.VMEM((HL,), jnp.float32),
        pltpu.SemaphoreType.DMA,
    ],
    compiler_params=pltpu.CompilerParams(
        use_tc_tiling_on_sc=False, needs_layout_passes=False),
)(_body)


def kernel(input):
    return _sparsemax_sc(input.reshape(N * C)).reshape(N, C)
